# baseline (speedup 1.0000x reference)
import functools

import jax
import jax.numpy as jnp
from jax import lax
from jax.experimental import pallas as pl
from jax.experimental.pallas import tpu as pltpu
from jax.experimental.pallas import tpu_sc as plsc

B = 2
N = 50000
K = 16
H = 128
NF = 10

NC = 2
NS = 16
NW = NC * NS

CHUNK = 1568
GROUPS = CHUNK // 16

NPAD = 51200
NB = 2048
NBLK = NPAD // NB
KP = K // 2


def _prep_body(*refs):
  idx_refs = refs[0:B]
  dist_refs = refs[B:2 * B]
  idxp_refs = refs[2 * B:2 * B + B * KP]
  dm_refs = refs[2 * B + B * KP:]
  for b in range(B):
    xt = jnp.transpose(idx_refs[b][...])
    w = jnp.bitwise_or(xt[0:KP], jnp.left_shift(xt[KP:K], 16))
    for t in range(KP):
      idxp_refs[b * KP + t][...] = w[t]
    dt = jnp.transpose(dist_refs[b][...])
    dm_refs[b][...] = jnp.max(dt, axis=0)


def _prep_call(neigh_idx, neigh_dist):
  blk_in = pl.BlockSpec((NB, K), lambda i: (i, 0))
  blk_row = pl.BlockSpec((NB,), lambda i: (i,))
  return pl.pallas_call(
      _prep_body,
      grid=(NBLK,),
      in_specs=[blk_in] * (2 * B),
      out_specs=[blk_row] * (B * KP + B),
      out_shape=[jax.ShapeDtypeStruct((NPAD,), jnp.int32)] * (B * KP) + [
          jax.ShapeDtypeStruct((NPAD,), jnp.float32)] * B,
  )(neigh_idx[0], neigh_idx[1], neigh_dist[0], neigh_dist[1])


def _pooled_body(*refs):
  pos_hbm = refs[0]
  idxp_hbm = refs[1:1 + B * KP]
  dm_hbm = refs[1 + B * KP:1 + B * KP + B]
  pooled_hbm = refs[1 + B * KP + B]
  plane0_v, plane1_v, idxp_v, pooled_v, sem = refs[2 + B * KP + B:]
  wid = lax.axis_index("s") * NC + lax.axis_index("c")
  start = jnp.minimum(wid * CHUNK, N - CHUNK).astype(jnp.int32)
  planes = (plane0_v, plane1_v)

  def plane_dma(p):
    b, c = divmod(p, 3)
    return pltpu.async_copy(
        pos_hbm.at[pl.ds((b * 3 + c) * N, N)], planes[p % 2], sem)

  pending = plane_dma(0)
  for b in range(B):
    for t in range(KP):
      pltpu.sync_copy(
          idxp_hbm[b * KP + t].at[pl.ds(start, CHUNK)],
          idxp_v.at[pl.ds(t * CHUNK, CHUNK)])
    pltpu.sync_copy(dm_hbm[b].at[pl.ds(start, CHUNK)],
                    pooled_v.at[pl.ds(9 * CHUNK, CHUNK)])
    for c in range(3):
      p = b * 3 + c
      plane_v = planes[p % 2]
      pending.wait()
      if p + 1 < 6:
        pending = plane_dma(p + 1)

      def group_body(g, carry, plane_v=plane_v, c=c):
        selfv = plane_v[pl.ds(start + g * 16, 16)]
        amax = None
        for t in range(KP):
          w = idxp_v[pl.ds(t * CHUNK + g * 16, 16)]
          ia = jnp.bitwise_and(w, 0xFFFF)
          ib = lax.shift_right_logical(w, 16)
          va = plsc.load_gather(plane_v, [ia])
          vb = plsc.load_gather(plane_v, [ib])
          hi = jnp.maximum(va, vb)
          lo = jnp.minimum(va, vb)
          if t == 0:
            amax, amin = hi, lo
          else:
            amax = jnp.maximum(amax, hi)
            amin = jnp.minimum(amin, lo)
        pooled_v[pl.ds(c * CHUNK + g * 16, 16)] = selfv
        pooled_v[pl.ds((3 + c) * CHUNK + g * 16, 16)] = amax
        pooled_v[pl.ds((6 + c) * CHUNK + g * 16, 16)] = selfv - amin
        return carry

      lax.fori_loop(0, GROUPS, group_body, 0)
    for f in range(NF):
      pltpu.sync_copy(
          pooled_v.at[pl.ds(f * CHUNK, CHUNK)],
          pooled_hbm.at[pl.ds(f * (B * NPAD) + b * NPAD + start, CHUNK)])


_pooled_call = functools.partial(
    pl.kernel,
    out_type=jax.ShapeDtypeStruct((NF * B * NPAD,), jnp.float32),
    mesh=plsc.VectorSubcoreMesh(core_axis_name="c", subcore_axis_name="s"),
    compiler_params=pltpu.CompilerParams(needs_layout_passes=False),
    scratch_types=[
        pltpu.VMEM((N,), jnp.float32),
        pltpu.VMEM((N,), jnp.float32),
        pltpu.VMEM((KP * CHUNK,), jnp.int32),
        pltpu.VMEM((NF * CHUNK,), jnp.float32),
        pltpu.SemaphoreType.DMA,
    ],
)(_pooled_body)


def _mlp_body(x_ref, w_ref, b_ref, o_ref):
  y = lax.dot_general(x_ref[...], w_ref[...], (((0,), (0,)), ((), ())),
                      preferred_element_type=jnp.float32)
  o_ref[0] = jnp.maximum(y + b_ref[...], 0.0)


def _mlp_call(xt, wt, b2):
  return pl.pallas_call(
      _mlp_body,
      grid=(B, NBLK),
      in_specs=[
          pl.BlockSpec((NF, NB), lambda b, i: (0, b * NBLK + i)),
          pl.BlockSpec((NF, H), lambda b, i: (0, 0)),
          pl.BlockSpec((1, H), lambda b, i: (0, 0)),
      ],
      out_specs=pl.BlockSpec((1, NB, H), lambda b, i: (b, i, 0)),
      out_shape=jax.ShapeDtypeStruct((B, N, H), jnp.float32),
  )(xt, wt, b2)


def kernel(pos, neigh_idx, neigh_dist, W1, b1):
  outs = _prep_call(neigh_idx, neigh_dist)
  pooled = _pooled_call(pos.reshape(B * 3 * N), *outs)
  xt = pooled.reshape(NF, B * NPAD)
  return _mlp_call(xt, W1.T, b1.reshape(1, H))


# speedup vs baseline: 1.0493x; 1.0493x over previous
import functools

import jax
import jax.numpy as jnp
from jax import lax
from jax.experimental import pallas as pl
from jax.experimental.pallas import tpu as pltpu
from jax.experimental.pallas import tpu_sc as plsc

B = 2
N = 50000
K = 16
H = 128
NF = 10

NC = 2
NS = 16
NW = NC * NS

CHUNK = 1568
GROUPS = CHUNK // 16

NPAD = 51200
NB = 2048
NBLK = NPAD // NB
KP = K // 2


def _prep_body(*refs):
  idx_refs = refs[0:B]
  dist_refs = refs[B:2 * B]
  idxp_refs = refs[2 * B:2 * B + B * KP]
  dm_refs = refs[2 * B + B * KP:]
  for b in range(B):
    xt = jnp.transpose(idx_refs[b][0])
    w = jnp.bitwise_or(xt[0:KP], jnp.left_shift(xt[KP:K], 16))
    for t in range(KP):
      idxp_refs[b * KP + t][...] = w[t]
    dt = jnp.transpose(dist_refs[b][0])
    dm_refs[b][...] = jnp.max(dt, axis=0)


def _prep_call(neigh_idx, neigh_dist):
  def blk_in(b):
    return pl.BlockSpec((1, NB, K), lambda i, b=b: (b, i, 0))
  blk_row = pl.BlockSpec((NB,), lambda i: (i,))
  return pl.pallas_call(
      _prep_body,
      grid=(NBLK,),
      in_specs=[blk_in(0), blk_in(1), blk_in(0), blk_in(1)],
      out_specs=[blk_row] * (B * KP + B),
      out_shape=[jax.ShapeDtypeStruct((NPAD,), jnp.int32)] * (B * KP) + [
          jax.ShapeDtypeStruct((NPAD,), jnp.float32)] * B,
  )(neigh_idx, neigh_idx, neigh_dist, neigh_dist)


def _pooled_body(*refs):
  pos_hbm = refs[0]
  idxp_hbm = refs[1:1 + B * KP]
  dm_hbm = refs[1 + B * KP:1 + B * KP + B]
  pooled_hbm = refs[1 + B * KP + B]
  plane0_v, plane1_v, idxp_v, pooled_v, sem = refs[2 + B * KP + B:]
  wid = lax.axis_index("s") * NC + lax.axis_index("c")
  start = jnp.minimum(wid * CHUNK, N - CHUNK).astype(jnp.int32)
  planes = (plane0_v, plane1_v)

  def plane_dma(p):
    b, c = divmod(p, 3)
    return pltpu.async_copy(
        pos_hbm.at[pl.ds((b * 3 + c) * N, N)], planes[p % 2], sem)

  pending = plane_dma(0)
  for b in range(B):
    for t in range(KP):
      pltpu.sync_copy(
          idxp_hbm[b * KP + t].at[pl.ds(start, CHUNK)],
          idxp_v.at[pl.ds(t * CHUNK, CHUNK)])
    pltpu.sync_copy(dm_hbm[b].at[pl.ds(start, CHUNK)],
                    pooled_v.at[pl.ds(9 * CHUNK, CHUNK)])
    for c in range(3):
      p = b * 3 + c
      plane_v = planes[p % 2]
      pending.wait()
      if p + 1 < 6:
        pending = plane_dma(p + 1)

      def group_body(g, carry, plane_v=plane_v, c=c):
        selfv = plane_v[pl.ds(start + g * 16, 16)]
        amax = None
        for t in range(KP):
          w = idxp_v[pl.ds(t * CHUNK + g * 16, 16)]
          ia = jnp.bitwise_and(w, 0xFFFF)
          ib = lax.shift_right_logical(w, 16)
          va = plsc.load_gather(plane_v, [ia])
          vb = plsc.load_gather(plane_v, [ib])
          hi = jnp.maximum(va, vb)
          lo = jnp.minimum(va, vb)
          if t == 0:
            amax, amin = hi, lo
          else:
            amax = jnp.maximum(amax, hi)
            amin = jnp.minimum(amin, lo)
        pooled_v[pl.ds(c * CHUNK + g * 16, 16)] = selfv
        pooled_v[pl.ds((3 + c) * CHUNK + g * 16, 16)] = amax
        pooled_v[pl.ds((6 + c) * CHUNK + g * 16, 16)] = selfv - amin
        return carry

      lax.fori_loop(0, GROUPS, group_body, 0)
    for f in range(NF):
      pltpu.sync_copy(
          pooled_v.at[pl.ds(f * CHUNK, CHUNK)],
          pooled_hbm.at[pl.ds(f * (B * NPAD) + b * NPAD + start, CHUNK)])


_pooled_call = functools.partial(
    pl.kernel,
    out_type=jax.ShapeDtypeStruct((NF * B * NPAD,), jnp.float32),
    mesh=plsc.VectorSubcoreMesh(core_axis_name="c", subcore_axis_name="s"),
    compiler_params=pltpu.CompilerParams(needs_layout_passes=False),
    scratch_types=[
        pltpu.VMEM((N,), jnp.float32),
        pltpu.VMEM((N,), jnp.float32),
        pltpu.VMEM((KP * CHUNK,), jnp.int32),
        pltpu.VMEM((NF * CHUNK,), jnp.float32),
        pltpu.SemaphoreType.DMA,
    ],
)(_pooled_body)


def _mlp_body(x_ref, w_ref, b_ref, o_ref):
  y = lax.dot_general(x_ref[...], w_ref[...], (((0,), (0,)), ((), ())),
                      preferred_element_type=jnp.float32)
  o_ref[0] = jnp.maximum(y + b_ref[...], 0.0)


def _mlp_call(xt, wt, b2):
  return pl.pallas_call(
      _mlp_body,
      grid=(B, NBLK),
      in_specs=[
          pl.BlockSpec((NF, NB), lambda b, i: (0, b * NBLK + i)),
          pl.BlockSpec((NF, H), lambda b, i: (0, 0)),
          pl.BlockSpec((1, H), lambda b, i: (0, 0)),
      ],
      out_specs=pl.BlockSpec((1, NB, H), lambda b, i: (b, i, 0)),
      out_shape=jax.ShapeDtypeStruct((B, N, H), jnp.float32),
  )(xt, wt, b2)


def kernel(pos, neigh_idx, neigh_dist, W1, b1):
  outs = _prep_call(neigh_idx, neigh_dist)
  pooled = _pooled_call(pos.reshape(B * 3 * N), *outs)
  xt = pooled.reshape(NF, B * NPAD)
  return _mlp_call(xt, W1.T, b1.reshape(1, H))


# speedup vs baseline: 1.0794x; 1.0287x over previous
import functools

import jax
import jax.numpy as jnp
from jax import lax
from jax.experimental import pallas as pl
from jax.experimental.pallas import tpu as pltpu
from jax.experimental.pallas import tpu_sc as plsc

B = 2
N = 50000
K = 16
H = 128
NF = 10

NC = 2
NS = 16
NW = NC * NS

CHUNK = 1568
GROUPS = CHUNK // 16

NPAD = 51200
NB = 2048
NBLK = NPAD // NB
KP = K // 2


def _prep_body(*refs):
  idx_refs = refs[0:B]
  dist_refs = refs[B:2 * B]
  idxp_refs = refs[2 * B:2 * B + B * KP]
  dm_refs = refs[2 * B + B * KP:]
  for b in range(B):
    xt = jnp.transpose(idx_refs[b][0])
    w = jnp.bitwise_or(xt[0:KP], jnp.left_shift(xt[KP:K], 16))
    for t in range(KP):
      idxp_refs[b * KP + t][...] = w[t]
    dt = jnp.transpose(dist_refs[b][0])
    dm_refs[b][...] = jnp.max(dt, axis=0)


def _prep_call(neigh_idx, neigh_dist):
  def blk_in(b):
    return pl.BlockSpec((1, NB, K), lambda i, b=b: (b, i, 0))
  blk_row = pl.BlockSpec((NB,), lambda i: (i,))
  return pl.pallas_call(
      _prep_body,
      grid=(NBLK,),
      in_specs=[blk_in(0), blk_in(1), blk_in(0), blk_in(1)],
      out_specs=[blk_row] * (B * KP + B),
      out_shape=[jax.ShapeDtypeStruct((NPAD,), jnp.int32)] * (B * KP) + [
          jax.ShapeDtypeStruct((NPAD,), jnp.float32)] * B,
  )(neigh_idx, neigh_idx, neigh_dist, neigh_dist)


def _pooled_body(*refs):
  pos_hbm = refs[0]
  idxp_hbm = refs[1:1 + B * KP]
  dm_hbm = refs[1 + B * KP:1 + B * KP + B]
  pooled_hbm = refs[1 + B * KP + B]
  (plane0_v, plane1_v, idxp_v, pooled_v,
   sem, sem_in, sem_out) = refs[2 + B * KP + B:]
  wid = lax.axis_index("s") * NC + lax.axis_index("c")
  start = jnp.minimum(wid * CHUNK, N - CHUNK).astype(jnp.int32)
  planes = (plane0_v, plane1_v)

  def plane_dma(p):
    b, c = divmod(p, 3)
    return pltpu.async_copy(
        pos_hbm.at[pl.ds((b * 3 + c) * N, N)], planes[p % 2], sem)

  pending = plane_dma(0)
  out_handles = []
  for b in range(B):
    stage = [
        pltpu.async_copy(
            idxp_hbm[b * KP + t].at[pl.ds(start, CHUNK)],
            idxp_v.at[pl.ds(t * CHUNK, CHUNK)], sem_in)
        for t in range(KP)
    ]
    for h in out_handles:
      h.wait()
    out_handles = []
    stage.append(
        pltpu.async_copy(dm_hbm[b].at[pl.ds(start, CHUNK)],
                         pooled_v.at[pl.ds(9 * CHUNK, CHUNK)], sem_in))
    for h in stage:
      h.wait()
    for c in range(3):
      p = b * 3 + c
      plane_v = planes[p % 2]
      pending.wait()
      if p + 1 < 6:
        pending = plane_dma(p + 1)

      def group_body(g, carry, plane_v=plane_v, c=c):
        selfv = plane_v[pl.ds(start + g * 16, 16)]
        amax = None
        for t in range(KP):
          w = idxp_v[pl.ds(t * CHUNK + g * 16, 16)]
          ia = jnp.bitwise_and(w, 0xFFFF)
          ib = lax.shift_right_logical(w, 16)
          va = plsc.load_gather(plane_v, [ia])
          vb = plsc.load_gather(plane_v, [ib])
          hi = jnp.maximum(va, vb)
          lo = jnp.minimum(va, vb)
          if t == 0:
            amax, amin = hi, lo
          else:
            amax = jnp.maximum(amax, hi)
            amin = jnp.minimum(amin, lo)
        pooled_v[pl.ds(c * CHUNK + g * 16, 16)] = selfv
        pooled_v[pl.ds((3 + c) * CHUNK + g * 16, 16)] = amax
        pooled_v[pl.ds((6 + c) * CHUNK + g * 16, 16)] = selfv - amin
        return carry

      lax.fori_loop(0, GROUPS, group_body, 0)
    out_handles = [
        pltpu.async_copy(
            pooled_v.at[pl.ds(f * CHUNK, CHUNK)],
            pooled_hbm.at[pl.ds(f * (B * NPAD) + b * NPAD + start, CHUNK)],
            sem_out)
        for f in range(NF)
    ]
  for h in out_handles:
    h.wait()


_pooled_call = functools.partial(
    pl.kernel,
    out_type=jax.ShapeDtypeStruct((NF * B * NPAD,), jnp.float32),
    mesh=plsc.VectorSubcoreMesh(core_axis_name="c", subcore_axis_name="s"),
    compiler_params=pltpu.CompilerParams(needs_layout_passes=False),
    scratch_types=[
        pltpu.VMEM((N,), jnp.float32),
        pltpu.VMEM((N,), jnp.float32),
        pltpu.VMEM((KP * CHUNK,), jnp.int32),
        pltpu.VMEM((NF * CHUNK,), jnp.float32),
        pltpu.SemaphoreType.DMA,
        pltpu.SemaphoreType.DMA,
        pltpu.SemaphoreType.DMA,
    ],
)(_pooled_body)


def _mlp_body(x_ref, w_ref, b_ref, o_ref):
  y = lax.dot_general(x_ref[...], w_ref[...], (((0,), (0,)), ((), ())),
                      preferred_element_type=jnp.float32)
  o_ref[0] = jnp.maximum(y + b_ref[...], 0.0)


def _mlp_call(xt, wt, b2):
  return pl.pallas_call(
      _mlp_body,
      grid=(B, NBLK),
      in_specs=[
          pl.BlockSpec((NF, NB), lambda b, i: (0, b * NBLK + i)),
          pl.BlockSpec((NF, H), lambda b, i: (0, 0)),
          pl.BlockSpec((1, H), lambda b, i: (0, 0)),
      ],
      out_specs=pl.BlockSpec((1, NB, H), lambda b, i: (b, i, 0)),
      out_shape=jax.ShapeDtypeStruct((B, N, H), jnp.float32),
  )(xt, wt, b2)


def kernel(pos, neigh_idx, neigh_dist, W1, b1):
  outs = _prep_call(neigh_idx, neigh_dist)
  pooled = _pooled_call(pos.reshape(B * 3 * N), *outs)
  xt = pooled.reshape(NF, B * NPAD)
  return _mlp_call(xt, W1.T, b1.reshape(1, H))


# speedup vs baseline: 1.0810x; 1.0015x over previous
import functools

import jax
import jax.numpy as jnp
from jax import lax
from jax.experimental import pallas as pl
from jax.experimental.pallas import tpu as pltpu
from jax.experimental.pallas import tpu_sc as plsc

B = 2
N = 50000
K = 16
H = 128
NF = 10

NC = 2
NS = 16
NW = NC * NS

CHUNK = 1568
GROUPS = CHUNK // 16

NPAD = 51200
NB = 2048
NBLK = NPAD // NB
KP = K // 2


def _prep_body(*refs):
  idx_ref, dist_ref = refs[0], refs[1]
  idxp_refs = refs[2:2 + B * KP]
  dm_refs = refs[2 + B * KP:]
  x = idx_ref[...]
  d = dist_ref[...]
  for b in range(B):
    xt = jnp.transpose(x[b])
    w = jnp.bitwise_or(xt[0:KP], jnp.left_shift(xt[KP:K], 16))
    for t in range(KP):
      idxp_refs[b * KP + t][...] = w[t]
    dt = jnp.transpose(d[b])
    dm_refs[b][...] = jnp.max(dt, axis=0)


def _prep_call(neigh_idx, neigh_dist):
  blk_in = pl.BlockSpec((B, NB, K), lambda i: (0, i, 0))
  blk_row = pl.BlockSpec((NB,), lambda i: (i,))
  return pl.pallas_call(
      _prep_body,
      grid=(NBLK,),
      in_specs=[blk_in, blk_in],
      out_specs=[blk_row] * (B * KP + B),
      out_shape=[jax.ShapeDtypeStruct((NPAD,), jnp.int32)] * (B * KP) + [
          jax.ShapeDtypeStruct((NPAD,), jnp.float32)] * B,
  )(neigh_idx, neigh_dist)


def _pooled_body(*refs):
  pos_hbm = refs[0]
  idxp_hbm = refs[1:1 + B * KP]
  dm_hbm = refs[1 + B * KP:1 + B * KP + B]
  pooled_hbm = refs[1 + B * KP + B]
  (plane0_v, plane1_v, idxp_v, pooled_v,
   sem, sem_in, sem_out) = refs[2 + B * KP + B:]
  wid = lax.axis_index("s") * NC + lax.axis_index("c")
  start = jnp.minimum(wid * CHUNK, N - CHUNK).astype(jnp.int32)
  planes = (plane0_v, plane1_v)

  def plane_dma(p):
    b, c = divmod(p, 3)
    return pltpu.async_copy(
        pos_hbm.at[pl.ds((b * 3 + c) * N, N)], planes[p % 2], sem)

  pending = plane_dma(0)
  out_handles = []
  for b in range(B):
    stage = [
        pltpu.async_copy(
            idxp_hbm[b * KP + t].at[pl.ds(start, CHUNK)],
            idxp_v.at[pl.ds(t * CHUNK, CHUNK)], sem_in)
        for t in range(KP)
    ]
    for h in out_handles:
      h.wait()
    out_handles = []
    stage.append(
        pltpu.async_copy(dm_hbm[b].at[pl.ds(start, CHUNK)],
                         pooled_v.at[pl.ds(9 * CHUNK, CHUNK)], sem_in))
    for h in stage:
      h.wait()
    for c in range(3):
      p = b * 3 + c
      plane_v = planes[p % 2]
      pending.wait()
      if p + 1 < 6:
        pending = plane_dma(p + 1)

      def group_body(g, carry, plane_v=plane_v, c=c):
        selfv = plane_v[pl.ds(start + g * 16, 16)]
        amax = None
        for t in range(KP):
          w = idxp_v[pl.ds(t * CHUNK + g * 16, 16)]
          ia = jnp.bitwise_and(w, 0xFFFF)
          ib = lax.shift_right_logical(w, 16)
          va = plsc.load_gather(plane_v, [ia])
          vb = plsc.load_gather(plane_v, [ib])
          hi = jnp.maximum(va, vb)
          lo = jnp.minimum(va, vb)
          if t == 0:
            amax, amin = hi, lo
          else:
            amax = jnp.maximum(amax, hi)
            amin = jnp.minimum(amin, lo)
        pooled_v[pl.ds(c * CHUNK + g * 16, 16)] = selfv
        pooled_v[pl.ds((3 + c) * CHUNK + g * 16, 16)] = amax
        pooled_v[pl.ds((6 + c) * CHUNK + g * 16, 16)] = selfv - amin
        return carry

      lax.fori_loop(0, GROUPS, group_body, 0)
    out_handles = [
        pltpu.async_copy(
            pooled_v.at[pl.ds(f * CHUNK, CHUNK)],
            pooled_hbm.at[pl.ds(f * (B * NPAD) + b * NPAD + start, CHUNK)],
            sem_out)
        for f in range(NF)
    ]
  for h in out_handles:
    h.wait()


_pooled_call = functools.partial(
    pl.kernel,
    out_type=jax.ShapeDtypeStruct((NF * B * NPAD,), jnp.float32),
    mesh=plsc.VectorSubcoreMesh(core_axis_name="c", subcore_axis_name="s"),
    compiler_params=pltpu.CompilerParams(needs_layout_passes=False),
    scratch_types=[
        pltpu.VMEM((N,), jnp.float32),
        pltpu.VMEM((N,), jnp.float32),
        pltpu.VMEM((KP * CHUNK,), jnp.int32),
        pltpu.VMEM((NF * CHUNK,), jnp.float32),
        pltpu.SemaphoreType.DMA,
        pltpu.SemaphoreType.DMA,
        pltpu.SemaphoreType.DMA,
    ],
)(_pooled_body)


def _mlp_body(x_ref, w_ref, b_ref, o_ref):
  y = lax.dot_general(x_ref[...], w_ref[...], (((0,), (0,)), ((), ())),
                      preferred_element_type=jnp.float32)
  o_ref[0] = jnp.maximum(y + b_ref[...], 0.0)


def _mlp_call(xt, wt, b2):
  return pl.pallas_call(
      _mlp_body,
      grid=(B, NBLK),
      in_specs=[
          pl.BlockSpec((NF, NB), lambda b, i: (0, b * NBLK + i)),
          pl.BlockSpec((NF, H), lambda b, i: (0, 0)),
          pl.BlockSpec((1, H), lambda b, i: (0, 0)),
      ],
      out_specs=pl.BlockSpec((1, NB, H), lambda b, i: (b, i, 0)),
      out_shape=jax.ShapeDtypeStruct((B, N, H), jnp.float32),
  )(xt, wt, b2)


def kernel(pos, neigh_idx, neigh_dist, W1, b1):
  outs = _prep_call(neigh_idx, neigh_dist)
  pooled = _pooled_call(pos.reshape(B * 3 * N), *outs)
  xt = pooled.reshape(NF, B * NPAD)
  return _mlp_call(xt, W1.T, b1.reshape(1, H))
